# initial kernel scaffold (unmeasured)
import jax
import jax.numpy as jnp
from jax import lax
from jax.experimental import pallas as pl
from jax.experimental.pallas import tpu as pltpu

N_DEV = 32
M_PER = 128
K = 4096
N = 8192
N_PER = 256
NBLK = 32
BLK_N = N // NBLK


def kernel(x, w_mat):
    def body(x_ref, w_ref, out_ref,
             y_ref, qsend_ref, qrecv_ref,
             amax_send_ref, amax_recv_ref,
             amax_send_sems, amax_recv_sems,
             a2a_send_sems, a2a_recv_sems):
        j = pl.program_id(0)
        my_i = lax.axis_index("i")

        yblk = jnp.dot(x_ref[:, :], w_ref[:, :],
                       preferred_element_type=jnp.float32)
        y_ref[:, pl.ds(j * BLK_N, BLK_N)] = jnp.maximum(yblk, 0.0)

        @pl.when(j == NBLK - 1)
        def _comm():
            amax_local = jnp.max(y_ref[:, :])
            tile = jnp.full((8, 128), amax_local, jnp.float32)
            amax_send_ref[:, :] = tile
            amax_recv_ref[my_i] = tile

            for d in range(1, N_DEV):
                tgt = lax.rem(my_i + d, N_DEV)
                pltpu.make_async_remote_copy(
                    src_ref=amax_send_ref,
                    dst_ref=amax_recv_ref.at[my_i],
                    send_sem=amax_send_sems.at[d],
                    recv_sem=amax_recv_sems.at[my_i],
                    device_id=(tgt,),
                    device_id_type=pl.DeviceIdType.MESH,
                ).start()
            for d in range(1, N_DEV):
                src_dev = lax.rem(my_i - d + N_DEV, N_DEV)
                pltpu.make_async_remote_copy(
                    src_ref=amax_send_ref,
                    dst_ref=amax_recv_ref.at[src_dev],
                    send_sem=amax_send_sems.at[0],
                    recv_sem=amax_recv_sems.at[src_dev],
                    device_id=(my_i,),
                    device_id_type=pl.DeviceIdType.MESH,
                ).wait_recv()

            amax = jnp.max(amax_recv_ref[:, :, :])
            scale = amax / 127.0

            for p in range(N_DEV):
                q = jnp.round(y_ref[:, pl.ds(p * N_PER, N_PER)] / scale)
                qsend_ref[p] = jnp.clip(q, -127.0, 127.0).astype(jnp.int8)

            for d in range(1, N_DEV):
                tgt = lax.rem(my_i + d, N_DEV)
                pltpu.make_async_remote_copy(
                    src_ref=qsend_ref.at[tgt],
                    dst_ref=qrecv_ref.at[my_i],
                    send_sem=a2a_send_sems.at[d],
                    recv_sem=a2a_recv_sems.at[my_i],
                    device_id=(tgt,),
                    device_id_type=pl.DeviceIdType.MESH,
                ).start()
            out_ref[pl.ds(my_i * M_PER, M_PER), :] = (
                qsend_ref[my_i].astype(jnp.float32) * scale)

            for d in range(1, N_DEV):
                src_dev = lax.rem(my_i - d + N_DEV, N_DEV)
                pltpu.make_async_remote_copy(
                    src_ref=qsend_ref.at[0],
                    dst_ref=qrecv_ref.at[src_dev],
                    send_sem=a2a_send_sems.at[0],
                    recv_sem=a2a_recv_sems.at[src_dev],
                    device_id=(my_i,),
                    device_id_type=pl.DeviceIdType.MESH,
                ).wait_recv()
                out_ref[pl.ds(src_dev * M_PER, M_PER), :] = (
                    qrecv_ref[src_dev].astype(jnp.float32) * scale)

            for d in range(1, N_DEV):
                pltpu.make_async_remote_copy(
                    src_ref=amax_send_ref,
                    dst_ref=amax_recv_ref.at[my_i],
                    send_sem=amax_send_sems.at[d],
                    recv_sem=amax_recv_sems.at[my_i],
                    device_id=(my_i,),
                    device_id_type=pl.DeviceIdType.MESH,
                ).wait_send()
                pltpu.make_async_remote_copy(
                    src_ref=qsend_ref.at[0],
                    dst_ref=qrecv_ref.at[my_i],
                    send_sem=a2a_send_sems.at[d],
                    recv_sem=a2a_recv_sems.at[my_i],
                    device_id=(my_i,),
                    device_id_type=pl.DeviceIdType.MESH,
                ).wait_send()

    return pl.pallas_call(
        body,
        grid=(NBLK,),
        out_shape=jax.ShapeDtypeStruct((N_DEV * M_PER, N_PER), jnp.float32),
        in_specs=[
            pl.BlockSpec((M_PER, K), lambda j: (0, 0)),
            pl.BlockSpec((K, BLK_N), lambda j: (0, j)),
        ],
        out_specs=pl.BlockSpec((N_DEV * M_PER, N_PER), lambda j: (0, 0)),
        scratch_shapes=[
            pltpu.VMEM((M_PER, N), jnp.float32),
            pltpu.VMEM((N_DEV, M_PER, N_PER), jnp.int8),
            pltpu.VMEM((N_DEV, M_PER, N_PER), jnp.int8),
            pltpu.VMEM((8, 128), jnp.float32),
            pltpu.VMEM((N_DEV, 8, 128), jnp.float32),
            pltpu.SemaphoreType.DMA((N_DEV,)),
            pltpu.SemaphoreType.DMA((N_DEV,)),
            pltpu.SemaphoreType.DMA((N_DEV,)),
            pltpu.SemaphoreType.DMA((N_DEV,)),
        ],
        compiler_params=pltpu.CompilerParams(
            dimension_semantics=("arbitrary",),
            collective_id=0,
        ),
    )(x, w_mat)


# baseline (device time: 84211 ns/iter reference)
import jax
import jax.numpy as jnp
from jax import lax
from jax.experimental import pallas as pl
from jax.experimental.pallas import tpu as pltpu

N_DEV = 32
M_PER = 128
K = 4096
N = 8192
N_PER = 256
NBLK = 32
BLK_N = N // NBLK


def kernel(x, w_mat):
    def body(x_ref, w_ref, out_ref,
             y_ref, qsend_ref, qrecv_ref,
             amax_send_ref, amax_recv_ref,
             amax_send_sems, amax_recv_sems,
             a2a_send_sems, a2a_recv_sems):
        j = pl.program_id(0)
        my_i = lax.axis_index("i")

        yblk = jnp.dot(x_ref[:, :], w_ref[:, :],
                       preferred_element_type=jnp.float32)
        y_ref[:, pl.ds(j * BLK_N, BLK_N)] = jnp.maximum(yblk, 0.0)

        @pl.when(j == NBLK - 1)
        def _comm():
            amax_local = jnp.max(y_ref[:, :])
            tile = jnp.full((8, 128), amax_local, jnp.float32)
            amax_send_ref[:, :] = tile
            amax_recv_ref[my_i] = tile

            for d in range(1, N_DEV):
                tgt = lax.rem(my_i + d, N_DEV)
                pltpu.make_async_remote_copy(
                    src_ref=amax_send_ref,
                    dst_ref=amax_recv_ref.at[my_i],
                    send_sem=amax_send_sems.at[d],
                    recv_sem=amax_recv_sems.at[my_i],
                    device_id=(tgt,),
                    device_id_type=pl.DeviceIdType.MESH,
                ).start()
            for d in range(1, N_DEV):
                src_dev = lax.rem(my_i - d + N_DEV, N_DEV)
                pltpu.make_async_remote_copy(
                    src_ref=amax_send_ref,
                    dst_ref=amax_recv_ref.at[src_dev],
                    send_sem=amax_send_sems.at[0],
                    recv_sem=amax_recv_sems.at[src_dev],
                    device_id=(my_i,),
                    device_id_type=pl.DeviceIdType.MESH,
                ).wait_recv()

            amax = jnp.max(amax_recv_ref[:, :, :])
            scale = amax / 127.0

            for p in range(N_DEV):
                q = jnp.round(y_ref[:, pl.ds(p * N_PER, N_PER)] / scale)
                qsend_ref[p] = jnp.clip(q, -127.0, 127.0).astype(jnp.int8)

            for d in range(1, N_DEV):
                tgt = lax.rem(my_i + d, N_DEV)
                pltpu.make_async_remote_copy(
                    src_ref=qsend_ref.at[tgt],
                    dst_ref=qrecv_ref.at[my_i],
                    send_sem=a2a_send_sems.at[d],
                    recv_sem=a2a_recv_sems.at[my_i],
                    device_id=(tgt,),
                    device_id_type=pl.DeviceIdType.MESH,
                ).start()
            out_ref[pl.ds(my_i * M_PER, M_PER), :] = (
                qsend_ref[my_i].astype(jnp.float32) * scale)

            for d in range(1, N_DEV):
                src_dev = lax.rem(my_i - d + N_DEV, N_DEV)
                pltpu.make_async_remote_copy(
                    src_ref=qsend_ref.at[0],
                    dst_ref=qrecv_ref.at[src_dev],
                    send_sem=a2a_send_sems.at[0],
                    recv_sem=a2a_recv_sems.at[src_dev],
                    device_id=(my_i,),
                    device_id_type=pl.DeviceIdType.MESH,
                ).wait_recv()
                out_ref[pl.ds(src_dev * M_PER, M_PER), :] = (
                    qrecv_ref[src_dev].astype(jnp.float32) * scale)

            for d in range(1, N_DEV):
                pltpu.make_async_remote_copy(
                    src_ref=amax_send_ref,
                    dst_ref=amax_recv_ref.at[my_i],
                    send_sem=amax_send_sems.at[d],
                    recv_sem=amax_recv_sems.at[my_i],
                    device_id=(my_i,),
                    device_id_type=pl.DeviceIdType.MESH,
                ).wait_send()
                pltpu.make_async_remote_copy(
                    src_ref=qsend_ref.at[0],
                    dst_ref=qrecv_ref.at[my_i],
                    send_sem=a2a_send_sems.at[d],
                    recv_sem=a2a_recv_sems.at[my_i],
                    device_id=(my_i,),
                    device_id_type=pl.DeviceIdType.MESH,
                ).wait_send()

    return pl.pallas_call(
        body,
        grid=(NBLK,),
        out_shape=jax.ShapeDtypeStruct((N_DEV * M_PER, N_PER), jnp.float32),
        in_specs=[
            pl.BlockSpec((M_PER, K), lambda j: (0, 0)),
            pl.BlockSpec((K, BLK_N), lambda j: (0, j)),
        ],
        out_specs=pl.BlockSpec((N_DEV * M_PER, N_PER), lambda j: (0, 0)),
        scratch_shapes=[
            pltpu.VMEM((M_PER, N), jnp.float32),
            pltpu.VMEM((N_DEV, M_PER, N_PER), jnp.int8),
            pltpu.VMEM((N_DEV, M_PER, N_PER), jnp.int8),
            pltpu.VMEM((8, 128), jnp.float32),
            pltpu.VMEM((N_DEV, 8, 128), jnp.float32),
            pltpu.SemaphoreType.DMA((N_DEV,)),
            pltpu.SemaphoreType.DMA((N_DEV,)),
            pltpu.SemaphoreType.DMA((N_DEV,)),
            pltpu.SemaphoreType.DMA((N_DEV,)),
        ],
        compiler_params=pltpu.CompilerParams(
            dimension_semantics=("arbitrary",),
        ),
    )(x, w_mat)
